# baseline (speedup 1.0000x reference)
import functools

import jax
import jax.numpy as jnp
import numpy as np
from jax import lax
from jax.experimental import pallas as pl
from jax.experimental.pallas import tpu as pltpu
from jax.experimental.pallas import tpu_sc as plsc

N = 10000
E = 320000
D_IN = 128
D_MODEL = 128
D_LAP = 8
H = 8
DH = D_MODEL // H
L = 2
INV_SCALE = 1.0 / float(np.sqrt(DH))

BN = 1000
NC = 2
NS = 16
NW = NC * NS
EPW = E // NW
CH = 40
NCHUNK = EPW // CH
N_AV = 10240
ND = 1280
N_TOT = N_AV + ND
RPS = N_TOT // NS


def _ln(x, g, b):
    mu = jnp.mean(x, axis=-1, keepdims=True)
    var = jnp.mean((x - mu) ** 2, axis=-1, keepdims=True)
    return (x - mu) / jnp.sqrt(var + 1e-5) * g + b


def _pre_kernel(feat_ref, lap_ref, We_ref, be_ref, Wl_ref, bl_ref, sign_ref,
                wq_ref, wk_ref, wv_ref, h_ref, q_ref, k_ref, v_ref):
    h = (feat_ref[...] @ We_ref[...] + be_ref[...]
         + (lap_ref[...] * sign_ref[...]) @ Wl_ref[...] + bl_ref[...])
    h_ref[...] = h
    q_ref[...] = h @ wq_ref[...]
    k_ref[...] = h @ wk_ref[...]
    v_ref[...] = h @ wv_ref[...]


def _tail_kernel(pav_ref, pd_ref, hin_ref, wo_ref, bo_ref, g1_ref, b1_ref,
                 wf1_ref, bf1_ref, wf2_ref, bf2_ref, g2_ref, b2_ref,
                 wq_ref, wk_ref, wv_ref,
                 h_ref, q_ref, k_ref, v_ref, *, last):
    av = pav_ref[0] + pav_ref[1]
    den = jnp.maximum(pd_ref[0] + pd_ref[1], 1e-20)
    hagg = (av.reshape(BN, H, DH) / den[:, :, None]).reshape(BN, D_MODEL)
    h = hagg @ wo_ref[...] + bo_ref[...]
    h = _ln(hin_ref[...] + h, g1_ref[...], b1_ref[...])
    h2 = jax.nn.relu(h @ wf1_ref[...] + bf1_ref[...]) @ wf2_ref[...] + bf2_ref[...]
    h = _ln(h + h2, g2_ref[...], b2_ref[...])
    h_ref[...] = h
    if not last:
        q_ref[...] = h @ wq_ref[...]
        k_ref[...] = h @ wk_ref[...]
        v_ref[...] = h @ wv_ref[...]


def _row_block(i):
    return (i, 0)


def _full(shape):
    return pl.BlockSpec(shape, lambda i: (0,) * len(shape))


def _pre(feat, lap_pos, We, be, Wl, bl, sign, wq, wk, wv):
    rb = pl.BlockSpec((BN, D_MODEL), _row_block)
    return pl.pallas_call(
        _pre_kernel,
        grid=(N // BN,),
        in_specs=[
            pl.BlockSpec((BN, D_IN), _row_block),
            pl.BlockSpec((BN, D_LAP), _row_block),
            _full((D_IN, D_MODEL)), _full((D_MODEL,)),
            _full((D_LAP, D_MODEL)), _full((D_MODEL,)),
            _full((D_LAP,)),
            _full((D_MODEL, D_MODEL)), _full((D_MODEL, D_MODEL)), _full((D_MODEL, D_MODEL)),
        ],
        out_specs=[rb, rb, rb, rb],
        out_shape=[jax.ShapeDtypeStruct((N, D_MODEL), jnp.float32)] * 4,
    )(feat, lap_pos, We, be, Wl, bl, sign, wq, wk, wv)


def _tail(part, hin, wo, bo, g1, b1, wf1, bf1, wf2, bf2, g2, b2, wq, wk, wv, last):
    partd = part[:, N_AV:, :].reshape(NC, N_AV, 16)[:, :, :H]
    rb = pl.BlockSpec((BN, D_MODEL), _row_block)
    return pl.pallas_call(
        functools.partial(_tail_kernel, last=last),
        grid=(N // BN,),
        in_specs=[
            pl.BlockSpec((NC, BN, D_MODEL), lambda i: (0, i, 0)),
            pl.BlockSpec((NC, BN, H), lambda i: (0, i, 0)),
            rb,
            _full((D_MODEL, D_MODEL)), _full((D_MODEL,)),
            _full((D_MODEL,)), _full((D_MODEL,)),
            _full((D_MODEL, 2 * D_MODEL)), _full((2 * D_MODEL,)),
            _full((2 * D_MODEL, D_MODEL)), _full((D_MODEL,)),
            _full((D_MODEL,)), _full((D_MODEL,)),
            _full((D_MODEL, D_MODEL)), _full((D_MODEL, D_MODEL)), _full((D_MODEL, D_MODEL)),
        ],
        out_specs=[rb, rb, rb, rb],
        out_shape=[jax.ShapeDtypeStruct((N, D_MODEL), jnp.float32)] * 4,
    )(part, partd, hin, wo, bo, g1, b1, wf1, bf1, wf2, bf2, g2, b2, wq, wk, wv)


def _edge_body(q_hbm, k_hbm, v_hbm, src_hbm, dst_hbm, z_hbm, part_hbm,
               idx_s0, idx_d0, idx_s1, idx_d1, idx_d2,
               krows0, qrows0, vrows0, krows1, qrows1, vrows1,
               dbuf, acc, sem0, sem1):
    cid = lax.axis_index("c")
    sid = lax.axis_index("s")
    wid = cid * NS + sid
    ebase = wid * EPW

    idx_s = [idx_s0, idx_s1]
    idx_d = [idx_d0, idx_d1]
    krows = [krows0, krows1]
    qrows = [qrows0, qrows1]
    vrows = [vrows0, vrows1]
    sems = [sem0, sem1]

    pltpu.sync_copy(z_hbm.at[pl.ds(sid * RPS, RPS)], acc.at[pl.ds(sid * RPS, RPS)])
    plsc.subcore_barrier()

    lane = lax.iota(jnp.int32, 16)
    rots = [(lane + sh) % 16 for sh in (8, 4, 2, 1)]
    lanef = lane.astype(jnp.float32)
    ohs = [jnp.maximum(1.0 - jnp.abs(lanef - float(h)), 0.0) for h in range(H)]
    jbc = [jnp.broadcast_to(jnp.int32(j), (16,)) for j in range(8)]

    def fire(t, b):
        base = ebase + t * CH
        pltpu.sync_copy(src_hbm.at[pl.ds(base, CH)], idx_s[b])
        pltpu.sync_copy(dst_hbm.at[pl.ds(base, CH)], idx_d[b])
        pltpu.async_copy(k_hbm.at[idx_s[b]], krows[b], sems[b])
        pltpu.async_copy(q_hbm.at[idx_d[b]], qrows[b], sems[b])
        pltpu.async_copy(v_hbm.at[idx_s[b]], vrows[b], sems[b])

    def drain(b):
        pltpu.make_async_copy(k_hbm.at[idx_s[b]], krows[b], sems[b]).wait()
        pltpu.make_async_copy(q_hbm.at[idx_d[b]], qrows[b], sems[b]).wait()
        pltpu.make_async_copy(v_hbm.at[idx_s[b]], vrows[b], sems[b]).wait()

    def compute(b):
        for j in range(CH // 16):
            dv = idx_d[b][pl.ds(j * 16, 16)]
            idx_d2[pl.ds(j * 16, 16)] = lax.shift_right_logical(dv, 3) + N_AV
        if CH % 16:
            dv = idx_d[b][pl.ds(CH - 16, 16)]
            idx_d2[pl.ds(CH - 16, 16)] = lax.shift_right_logical(dv, 3) + N_AV

        kr, qr, vr, idd = krows[b], qrows[b], vrows[b], idx_d[b]

        def edge(e, carry2):
            srow = jnp.zeros((16,), jnp.float32)
            for h in range(H):
                kseg = kr[e, pl.ds(h * DH, DH)]
                qseg = qr[e, pl.ds(h * DH, DH)]
                t2 = kseg * qseg
                for r in rots:
                    t2 = t2 + t2[r]
                srow = srow + t2 * ohs[h]
            sc = jnp.clip(srow * INV_SCALE, -5.0, 5.0)
            ee = jnp.exp(sc)
            arow = jnp.exp(jnp.minimum(ee, 80.0))
            for h in range(H):
                vseg = vr[e, pl.ds(h * DH, DH)]
                vr[e, pl.ds(h * DH, DH)] = vseg * arow[jbc[h]]
            dvec = idd[pl.ds((e >> 4) << 4, 16)]
            j0f = ((dvec & 7)[jnp.broadcast_to(e & 15, (16,))]).astype(jnp.float32)
            zo = jnp.maximum(1.0 - jnp.abs(lanef - j0f), 0.0)
            for j in range(8):
                dbuf[e, pl.ds(j * 16, 16)] = arow * zo[jbc[j]]
            return carry2

        lax.fori_loop(0, CH, edge, 0, unroll=20)
        pltpu.sync_copy(vr, acc.at[idd], add=True)
        pltpu.sync_copy(dbuf, acc.at[idx_d2], add=True)

    fire(0, 0)

    def pair(g, carry):
        t0 = g * 2
        for b in range(2):
            t = t0 + b
            nxt = 1 - b

            @pl.when(t + 1 < NCHUNK)
            def _():
                fire(t + 1, nxt)

            drain(b)
            compute(b)
        return carry

    lax.fori_loop(0, NCHUNK // 2, pair, 0, unroll=1)

    plsc.subcore_barrier()
    pltpu.sync_copy(acc.at[pl.ds(sid * RPS, RPS)],
                    part_hbm.at[cid, pl.ds(sid * RPS, RPS)])


def _edge_sc(q, k, v, src, dst, zeros_acc):
    mesh = plsc.VectorSubcoreMesh(core_axis_name="c", subcore_axis_name="s")
    fn = pl.kernel(
        _edge_body,
        out_type=jax.ShapeDtypeStruct((NC, N_TOT, D_MODEL), jnp.float32),
        mesh=mesh,
        scratch_types=[
            pltpu.VMEM((CH,), jnp.int32),
            pltpu.VMEM((CH,), jnp.int32),
            pltpu.VMEM((CH,), jnp.int32),
            pltpu.VMEM((CH,), jnp.int32),
            pltpu.VMEM((CH,), jnp.int32),
            pltpu.VMEM((CH, D_MODEL), jnp.float32),
            pltpu.VMEM((CH, D_MODEL), jnp.float32),
            pltpu.VMEM((CH, D_MODEL), jnp.float32),
            pltpu.VMEM((CH, D_MODEL), jnp.float32),
            pltpu.VMEM((CH, D_MODEL), jnp.float32),
            pltpu.VMEM((CH, D_MODEL), jnp.float32),
            pltpu.VMEM((CH, D_MODEL), jnp.float32),
            pltpu.VMEM_SHARED((N_TOT, D_MODEL), jnp.float32),
            pltpu.SemaphoreType.DMA,
            pltpu.SemaphoreType.DMA,
        ],
    )
    return fn(q, k, v, src, dst, zeros_acc)


def kernel(feat, lap_pos, edge_index, We, be, Wl, bl, Wq, Wk, Wv, Wo, bo,
           g1, b1, Wf1, bf1, Wf2, bf2, g2, b2):
    src = edge_index[0]
    dst = edge_index[1]
    sf = jax.random.uniform(jax.random.key(123), (D_LAP,), dtype=jnp.float32)
    sign = jnp.where(sf >= 0.5, 1.0, -1.0)
    zeros_acc = jnp.zeros((N_TOT, D_MODEL), jnp.float32)

    h, q, k, v = _pre(feat, lap_pos, We, be, Wl, bl, sign, Wq[0], Wk[0], Wv[0])
    for l in range(L):
        part = _edge_sc(q, k, v, src, dst, zeros_acc)
        last = l == L - 1
        nq = Wq[l + 1] if not last else Wq[l]
        nk = Wk[l + 1] if not last else Wk[l]
        nv = Wv[l + 1] if not last else Wv[l]
        h, q, k, v = _tail(part, h, Wo[l], bo[l], g1[l], b1[l], Wf1[l], bf1[l],
                           Wf2[l], bf2[l], g2[l], b2[l], nq, nk, nv, last)
    return jnp.mean(h, axis=0, keepdims=True)


# speedup vs baseline: 1.8830x; 1.8830x over previous
import functools

import jax
import jax.numpy as jnp
import numpy as np
from jax import lax
from jax.experimental import pallas as pl
from jax.experimental.pallas import tpu as pltpu
from jax.experimental.pallas import tpu_sc as plsc

N = 10000
E = 320000
D_IN = 128
D_MODEL = 128
D_LAP = 8
H = 8
DH = D_MODEL // H
L = 2
INV_SCALE = 1.0 / float(np.sqrt(DH))

BN = 1000
NC = 2
NS = 16
NW = NC * NS
EPW = E // NW
CH = 40
NCHUNK = EPW // CH
N_AV = 10240
ND = 1280
N_TOT = N_AV + ND
RPS = N_TOT // NS


def _ln(x, g, b):
    mu = jnp.mean(x, axis=-1, keepdims=True)
    var = jnp.mean((x - mu) ** 2, axis=-1, keepdims=True)
    return (x - mu) / jnp.sqrt(var + 1e-5) * g + b


def _pre_kernel(feat_ref, lap_ref, We_ref, be_ref, Wl_ref, bl_ref, sign_ref,
                wq_ref, wk_ref, wv_ref, h_ref, q_ref, k_ref, v_ref):
    h = (feat_ref[...] @ We_ref[...] + be_ref[...]
         + (lap_ref[...] * sign_ref[...]) @ Wl_ref[...] + bl_ref[...])
    h_ref[...] = h
    q_ref[...] = h @ wq_ref[...]
    k_ref[...] = h @ wk_ref[...]
    v_ref[...] = h @ wv_ref[...]


def _tail_kernel(pav_ref, pd_ref, hin_ref, wo_ref, bo_ref, g1_ref, b1_ref,
                 wf1_ref, bf1_ref, wf2_ref, bf2_ref, g2_ref, b2_ref,
                 wq_ref, wk_ref, wv_ref,
                 h_ref, q_ref, k_ref, v_ref, *, last):
    av = pav_ref[0] + pav_ref[1]
    den = jnp.maximum(pd_ref[0] + pd_ref[1], 1e-20)
    hagg = (av.reshape(BN, H, DH) / den[:, :, None]).reshape(BN, D_MODEL)
    h = hagg @ wo_ref[...] + bo_ref[...]
    h = _ln(hin_ref[...] + h, g1_ref[...], b1_ref[...])
    h2 = jax.nn.relu(h @ wf1_ref[...] + bf1_ref[...]) @ wf2_ref[...] + bf2_ref[...]
    h = _ln(h + h2, g2_ref[...], b2_ref[...])
    h_ref[...] = h
    if not last:
        q_ref[...] = h @ wq_ref[...]
        k_ref[...] = h @ wk_ref[...]
        v_ref[...] = h @ wv_ref[...]


def _row_block(i):
    return (i, 0)


def _full(shape):
    return pl.BlockSpec(shape, lambda i: (0,) * len(shape))


def _pre(feat, lap_pos, We, be, Wl, bl, sign, wq, wk, wv):
    rb = pl.BlockSpec((BN, D_MODEL), _row_block)
    return pl.pallas_call(
        _pre_kernel,
        grid=(N // BN,),
        in_specs=[
            pl.BlockSpec((BN, D_IN), _row_block),
            pl.BlockSpec((BN, D_LAP), _row_block),
            _full((D_IN, D_MODEL)), _full((D_MODEL,)),
            _full((D_LAP, D_MODEL)), _full((D_MODEL,)),
            _full((D_LAP,)),
            _full((D_MODEL, D_MODEL)), _full((D_MODEL, D_MODEL)), _full((D_MODEL, D_MODEL)),
        ],
        out_specs=[rb, rb, rb, rb],
        out_shape=[jax.ShapeDtypeStruct((N, D_MODEL), jnp.float32)] * 4,
    )(feat, lap_pos, We, be, Wl, bl, sign, wq, wk, wv)


def _tail(part, hin, wo, bo, g1, b1, wf1, bf1, wf2, bf2, g2, b2, wq, wk, wv, last):
    partd = part[:, N_AV:, :].reshape(NC, N_AV, 16)[:, :, :H]
    rb = pl.BlockSpec((BN, D_MODEL), _row_block)
    return pl.pallas_call(
        functools.partial(_tail_kernel, last=last),
        grid=(N // BN,),
        in_specs=[
            pl.BlockSpec((NC, BN, D_MODEL), lambda i: (0, i, 0)),
            pl.BlockSpec((NC, BN, H), lambda i: (0, i, 0)),
            rb,
            _full((D_MODEL, D_MODEL)), _full((D_MODEL,)),
            _full((D_MODEL,)), _full((D_MODEL,)),
            _full((D_MODEL, 2 * D_MODEL)), _full((2 * D_MODEL,)),
            _full((2 * D_MODEL, D_MODEL)), _full((D_MODEL,)),
            _full((D_MODEL,)), _full((D_MODEL,)),
            _full((D_MODEL, D_MODEL)), _full((D_MODEL, D_MODEL)), _full((D_MODEL, D_MODEL)),
        ],
        out_specs=[rb, rb, rb, rb],
        out_shape=[jax.ShapeDtypeStruct((N, D_MODEL), jnp.float32)] * 4,
    )(part, partd, hin, wo, bo, g1, b1, wf1, bf1, wf2, bf2, g2, b2, wq, wk, wv)


def _edge_body(q_hbm, k_hbm, v_hbm, src_hbm, dst_hbm, z_hbm, part_hbm,
               idx_s0, idx_d0, idx_s1, idx_d1, idx_d2,
               krows0, qrows0, vrows0, krows1, qrows1, vrows1,
               dbuf, acc, sem0, sem1):
    cid = lax.axis_index("c")
    sid = lax.axis_index("s")
    wid = cid * NS + sid
    ebase = wid * EPW

    idx_s = [idx_s0, idx_s1]
    idx_d = [idx_d0, idx_d1]
    krows = [krows0, krows1]
    qrows = [qrows0, qrows1]
    vrows = [vrows0, vrows1]
    sems = [sem0, sem1]

    pltpu.sync_copy(z_hbm.at[pl.ds(sid * RPS, RPS)], acc.at[pl.ds(sid * RPS, RPS)])
    plsc.subcore_barrier()

    lane = lax.iota(jnp.int32, 16)
    rots = [(lane + sh) % 16 for sh in (8, 4, 2, 1)]
    lanef = lane.astype(jnp.float32)
    ohs = [jnp.maximum(1.0 - jnp.abs(lanef - float(h)), 0.0) for h in range(H)]
    jbc = [jnp.broadcast_to(jnp.int32(j), (16,)) for j in range(8)]

    def fire(t, b):
        base = ebase + t * CH
        pltpu.sync_copy(src_hbm.at[pl.ds(base, CH)], idx_s[b])
        pltpu.sync_copy(dst_hbm.at[pl.ds(base, CH)], idx_d[b])
        pltpu.async_copy(k_hbm.at[idx_s[b]], krows[b], sems[b])
        pltpu.async_copy(q_hbm.at[idx_d[b]], qrows[b], sems[b])
        pltpu.async_copy(v_hbm.at[idx_s[b]], vrows[b], sems[b])

    def drain(b):
        pltpu.make_async_copy(k_hbm.at[idx_s[b]], krows[b], sems[b]).wait()
        pltpu.make_async_copy(q_hbm.at[idx_d[b]], qrows[b], sems[b]).wait()
        pltpu.make_async_copy(v_hbm.at[idx_s[b]], vrows[b], sems[b]).wait()

    def compute(b):
        for j in range(CH // 16):
            dv = idx_d[b][pl.ds(j * 16, 16)]
            idx_d2[pl.ds(j * 16, 16)] = lax.shift_right_logical(dv, 3) + N_AV
        if CH % 16:
            dv = idx_d[b][pl.ds(CH - 16, 16)]
            idx_d2[pl.ds(CH - 16, 16)] = lax.shift_right_logical(dv, 3) + N_AV

        kr, qr, vr, idd = krows[b], qrows[b], vrows[b], idx_d[b]

        def edge(e, carry2):
            srow = jnp.zeros((16,), jnp.float32)
            for h in range(H):
                kseg = kr[e, pl.ds(h * DH, DH)]
                qseg = qr[e, pl.ds(h * DH, DH)]
                t2 = kseg * qseg
                for r in rots:
                    t2 = t2 + t2[r]
                srow = srow + t2 * ohs[h]
            sc = jnp.clip(srow * INV_SCALE, -5.0, 5.0)
            ee = jnp.exp(sc)
            arow = jnp.exp(jnp.minimum(ee, 80.0))
            for h in range(H):
                vseg = vr[e, pl.ds(h * DH, DH)]
                vr[e, pl.ds(h * DH, DH)] = vseg * arow[jbc[h]]
            dvec = idd[pl.ds((e >> 4) << 4, 16)]
            j0f = ((dvec & 7)[jnp.broadcast_to(e & 15, (16,))]).astype(jnp.float32)
            zo = jnp.maximum(1.0 - jnp.abs(lanef - j0f), 0.0)
            for j in range(8):
                dbuf[e, pl.ds(j * 16, 16)] = arow * zo[jbc[j]]
            return carry2

        lax.fori_loop(0, CH, edge, 0, unroll=10)
        pltpu.sync_copy(vr, acc.at[idd], add=True)
        pltpu.sync_copy(dbuf, acc.at[idx_d2], add=True)

    fire(0, 0)

    def pair(g, carry):
        t0 = g * 2
        for b in range(2):
            t = t0 + b
            nxt = 1 - b

            @pl.when(t + 1 < NCHUNK)
            def _():
                fire(t + 1, nxt)

            drain(b)
            compute(b)
        return carry

    lax.fori_loop(0, NCHUNK // 2, pair, 0, unroll=1)

    plsc.subcore_barrier()
    pltpu.sync_copy(acc.at[pl.ds(sid * RPS, RPS)],
                    part_hbm.at[cid, pl.ds(sid * RPS, RPS)])


def _edge_sc(q, k, v, src, dst, zeros_acc):
    mesh = plsc.VectorSubcoreMesh(core_axis_name="c", subcore_axis_name="s")
    fn = pl.kernel(
        _edge_body,
        out_type=jax.ShapeDtypeStruct((NC, N_TOT, D_MODEL), jnp.float32),
        mesh=mesh,
        scratch_types=[
            pltpu.VMEM((CH,), jnp.int32),
            pltpu.VMEM((CH,), jnp.int32),
            pltpu.VMEM((CH,), jnp.int32),
            pltpu.VMEM((CH,), jnp.int32),
            pltpu.VMEM((CH,), jnp.int32),
            pltpu.VMEM((CH, D_MODEL), jnp.float32),
            pltpu.VMEM((CH, D_MODEL), jnp.float32),
            pltpu.VMEM((CH, D_MODEL), jnp.float32),
            pltpu.VMEM((CH, D_MODEL), jnp.float32),
            pltpu.VMEM((CH, D_MODEL), jnp.float32),
            pltpu.VMEM((CH, D_MODEL), jnp.float32),
            pltpu.VMEM((CH, D_MODEL), jnp.float32),
            pltpu.VMEM_SHARED((N_TOT, D_MODEL), jnp.float32),
            pltpu.SemaphoreType.DMA,
            pltpu.SemaphoreType.DMA,
        ],
    )
    return fn(q, k, v, src, dst, zeros_acc)


def kernel(feat, lap_pos, edge_index, We, be, Wl, bl, Wq, Wk, Wv, Wo, bo,
           g1, b1, Wf1, bf1, Wf2, bf2, g2, b2):
    src = edge_index[0]
    dst = edge_index[1]
    sf = jax.random.uniform(jax.random.key(123), (D_LAP,), dtype=jnp.float32)
    sign = jnp.where(sf >= 0.5, 1.0, -1.0)
    zeros_acc = jnp.zeros((N_TOT, D_MODEL), jnp.float32)

    h, q, k, v = _pre(feat, lap_pos, We, be, Wl, bl, sign, Wq[0], Wk[0], Wv[0])
    for l in range(L):
        part = _edge_sc(q, k, v, src, dst, zeros_acc)
        last = l == L - 1
        nq = Wq[l + 1] if not last else Wq[l]
        nk = Wk[l + 1] if not last else Wk[l]
        nv = Wv[l + 1] if not last else Wv[l]
        h, q, k, v = _tail(part, h, Wo[l], bo[l], g1[l], b1[l], Wf1[l], bf1[l],
                           Wf2[l], bf2[l], g2[l], b2[l], nq, nk, nv, last)
    return jnp.mean(h, axis=0, keepdims=True)


# speedup vs baseline: 1.9562x; 1.0389x over previous
import functools

import jax
import jax.numpy as jnp
import numpy as np
from jax import lax
from jax.experimental import pallas as pl
from jax.experimental.pallas import tpu as pltpu
from jax.experimental.pallas import tpu_sc as plsc

N = 10000
E = 320000
D_IN = 128
D_MODEL = 128
D_LAP = 8
H = 8
DH = D_MODEL // H
L = 2
INV_SCALE = 1.0 / float(np.sqrt(DH))

BN = 1000
NC = 2
NS = 16
NW = NC * NS
EPW = E // NW
CH = 40
NCHUNK = EPW // CH
N_AV = 10240
ND = 1280
N_TOT = N_AV + ND
RPS = N_TOT // NS


def _ln(x, g, b):
    mu = jnp.mean(x, axis=-1, keepdims=True)
    var = jnp.mean((x - mu) ** 2, axis=-1, keepdims=True)
    return (x - mu) / jnp.sqrt(var + 1e-5) * g + b


def _pre_kernel(feat_ref, lap_ref, We_ref, be_ref, Wl_ref, bl_ref, sign_ref,
                wq_ref, wk_ref, wv_ref, h_ref, q_ref, k_ref, v_ref):
    h = (feat_ref[...] @ We_ref[...] + be_ref[...]
         + (lap_ref[...] * sign_ref[...]) @ Wl_ref[...] + bl_ref[...])
    h_ref[...] = h
    q_ref[...] = h @ wq_ref[...]
    k_ref[...] = h @ wk_ref[...]
    v_ref[...] = h @ wv_ref[...]


def _tail_kernel(pav_ref, pd_ref, hin_ref, wo_ref, bo_ref, g1_ref, b1_ref,
                 wf1_ref, bf1_ref, wf2_ref, bf2_ref, g2_ref, b2_ref,
                 wq_ref, wk_ref, wv_ref,
                 h_ref, q_ref, k_ref, v_ref, *, last):
    av = pav_ref[0] + pav_ref[1]
    den = jnp.maximum(pd_ref[0] + pd_ref[1], 1e-20)
    hagg = (av.reshape(BN, H, DH) / den[:, :, None]).reshape(BN, D_MODEL)
    h = hagg @ wo_ref[...] + bo_ref[...]
    h = _ln(hin_ref[...] + h, g1_ref[...], b1_ref[...])
    h2 = jax.nn.relu(h @ wf1_ref[...] + bf1_ref[...]) @ wf2_ref[...] + bf2_ref[...]
    h = _ln(h + h2, g2_ref[...], b2_ref[...])
    h_ref[...] = h
    if not last:
        q_ref[...] = h @ wq_ref[...]
        k_ref[...] = h @ wk_ref[...]
        v_ref[...] = h @ wv_ref[...]


def _row_block(i):
    return (i, 0)


def _full(shape):
    return pl.BlockSpec(shape, lambda i: (0,) * len(shape))


def _pre(feat, lap_pos, We, be, Wl, bl, sign, wq, wk, wv):
    rb = pl.BlockSpec((BN, D_MODEL), _row_block)
    return pl.pallas_call(
        _pre_kernel,
        grid=(N // BN,),
        in_specs=[
            pl.BlockSpec((BN, D_IN), _row_block),
            pl.BlockSpec((BN, D_LAP), _row_block),
            _full((D_IN, D_MODEL)), _full((D_MODEL,)),
            _full((D_LAP, D_MODEL)), _full((D_MODEL,)),
            _full((D_LAP,)),
            _full((D_MODEL, D_MODEL)), _full((D_MODEL, D_MODEL)), _full((D_MODEL, D_MODEL)),
        ],
        out_specs=[rb, rb, rb, rb],
        out_shape=[jax.ShapeDtypeStruct((N, D_MODEL), jnp.float32)] * 4,
    )(feat, lap_pos, We, be, Wl, bl, sign, wq, wk, wv)


def _tail(part, hin, wo, bo, g1, b1, wf1, bf1, wf2, bf2, g2, b2, wq, wk, wv, last):
    partd = part[:, N_AV:, :].reshape(NC, N_AV, 16)[:, :, :H]
    rb = pl.BlockSpec((BN, D_MODEL), _row_block)
    return pl.pallas_call(
        functools.partial(_tail_kernel, last=last),
        grid=(N // BN,),
        in_specs=[
            pl.BlockSpec((NC, BN, D_MODEL), lambda i: (0, i, 0)),
            pl.BlockSpec((NC, BN, H), lambda i: (0, i, 0)),
            rb,
            _full((D_MODEL, D_MODEL)), _full((D_MODEL,)),
            _full((D_MODEL,)), _full((D_MODEL,)),
            _full((D_MODEL, 2 * D_MODEL)), _full((2 * D_MODEL,)),
            _full((2 * D_MODEL, D_MODEL)), _full((D_MODEL,)),
            _full((D_MODEL,)), _full((D_MODEL,)),
            _full((D_MODEL, D_MODEL)), _full((D_MODEL, D_MODEL)), _full((D_MODEL, D_MODEL)),
        ],
        out_specs=[rb, rb, rb, rb],
        out_shape=[jax.ShapeDtypeStruct((N, D_MODEL), jnp.float32)] * 4,
    )(part, partd, hin, wo, bo, g1, b1, wf1, bf1, wf2, bf2, g2, b2, wq, wk, wv)


def _edge_body(q_hbm, k_hbm, v_hbm, src_hbm, dst_hbm, z_hbm, part_hbm,
               idx_s0, idx_d0, idx_s1, idx_d1, idx_d2,
               krows0, qrows0, vrows0, krows1, qrows1, vrows1,
               dbuf, acc, sem0, sem1):
    cid = lax.axis_index("c")
    sid = lax.axis_index("s")
    wid = cid * NS + sid
    ebase = wid * EPW

    idx_s = [idx_s0, idx_s1]
    idx_d = [idx_d0, idx_d1]
    krows = [krows0, krows1]
    qrows = [qrows0, qrows1]
    vrows = [vrows0, vrows1]
    sems = [sem0, sem1]

    pltpu.sync_copy(z_hbm.at[pl.ds(sid * RPS, RPS)], acc.at[pl.ds(sid * RPS, RPS)])
    plsc.subcore_barrier()

    lane = lax.iota(jnp.int32, 16)
    rots = [(lane + sh) % 16 for sh in (8, 4, 2, 1)]
    lanef = lane.astype(jnp.float32)
    ohs = [jnp.maximum(1.0 - jnp.abs(lanef - float(h)), 0.0) for h in range(H)]
    jbc = [jnp.broadcast_to(jnp.int32(j), (16,)) for j in range(8)]

    def fire(t, b):
        base = ebase + t * CH
        pltpu.sync_copy(src_hbm.at[pl.ds(base, CH)], idx_s[b])
        pltpu.sync_copy(dst_hbm.at[pl.ds(base, CH)], idx_d[b])
        pltpu.async_copy(k_hbm.at[idx_s[b]], krows[b], sems[b])
        pltpu.async_copy(q_hbm.at[idx_d[b]], qrows[b], sems[b])
        pltpu.async_copy(v_hbm.at[idx_s[b]], vrows[b], sems[b])

    def drain(b):
        pltpu.make_async_copy(k_hbm.at[idx_s[b]], krows[b], sems[b]).wait()
        pltpu.make_async_copy(q_hbm.at[idx_d[b]], qrows[b], sems[b]).wait()
        pltpu.make_async_copy(v_hbm.at[idx_s[b]], vrows[b], sems[b]).wait()

    def compute(b):
        for j in range(CH // 16):
            dv = idx_d[b][pl.ds(j * 16, 16)]
            idx_d2[pl.ds(j * 16, 16)] = lax.shift_right_logical(dv, 3) + N_AV
        if CH % 16:
            dv = idx_d[b][pl.ds(CH - 16, 16)]
            idx_d2[pl.ds(CH - 16, 16)] = lax.shift_right_logical(dv, 3) + N_AV

        kr, qr, vr, idd = krows[b], qrows[b], vrows[b], idx_d[b]

        def edge(e, carry2):
            srow = jnp.zeros((16,), jnp.float32)
            for h in range(H):
                kseg = kr[e, pl.ds(h * DH, DH)]
                qseg = qr[e, pl.ds(h * DH, DH)]
                t2 = kseg * qseg
                for r in rots:
                    t2 = t2 + t2[r]
                srow = srow + t2 * ohs[h]
            sc = jnp.clip(srow * INV_SCALE, -5.0, 5.0)
            ee = jnp.exp(sc)
            arow = jnp.exp(jnp.minimum(ee, 80.0))
            for h in range(H):
                vseg = vr[e, pl.ds(h * DH, DH)]
                vr[e, pl.ds(h * DH, DH)] = vseg * arow[jbc[h]]
            dvec = idd[pl.ds((e >> 4) << 4, 16)]
            j0f = ((dvec & 7)[jnp.broadcast_to(e & 15, (16,))]).astype(jnp.float32)
            zo = jnp.maximum(1.0 - jnp.abs(lanef - j0f), 0.0)
            for j in range(8):
                dbuf[e, pl.ds(j * 16, 16)] = arow * zo[jbc[j]]
            return carry2

        lax.fori_loop(0, CH, edge, 0, unroll=8)
        pltpu.sync_copy(vr, acc.at[idd], add=True)
        pltpu.sync_copy(dbuf, acc.at[idx_d2], add=True)

    fire(0, 0)

    def pair(g, carry):
        t0 = g * 2
        for b in range(2):
            t = t0 + b
            nxt = 1 - b

            @pl.when(t + 1 < NCHUNK)
            def _():
                fire(t + 1, nxt)

            drain(b)
            compute(b)
        return carry

    lax.fori_loop(0, NCHUNK // 2, pair, 0, unroll=1)

    plsc.subcore_barrier()
    pltpu.sync_copy(acc.at[pl.ds(sid * RPS, RPS)],
                    part_hbm.at[cid, pl.ds(sid * RPS, RPS)])


def _edge_sc(q, k, v, src, dst, zeros_acc):
    mesh = plsc.VectorSubcoreMesh(core_axis_name="c", subcore_axis_name="s")
    fn = pl.kernel(
        _edge_body,
        out_type=jax.ShapeDtypeStruct((NC, N_TOT, D_MODEL), jnp.float32),
        mesh=mesh,
        scratch_types=[
            pltpu.VMEM((CH,), jnp.int32),
            pltpu.VMEM((CH,), jnp.int32),
            pltpu.VMEM((CH,), jnp.int32),
            pltpu.VMEM((CH,), jnp.int32),
            pltpu.VMEM((CH,), jnp.int32),
            pltpu.VMEM((CH, D_MODEL), jnp.float32),
            pltpu.VMEM((CH, D_MODEL), jnp.float32),
            pltpu.VMEM((CH, D_MODEL), jnp.float32),
            pltpu.VMEM((CH, D_MODEL), jnp.float32),
            pltpu.VMEM((CH, D_MODEL), jnp.float32),
            pltpu.VMEM((CH, D_MODEL), jnp.float32),
            pltpu.VMEM((CH, D_MODEL), jnp.float32),
            pltpu.VMEM_SHARED((N_TOT, D_MODEL), jnp.float32),
            pltpu.SemaphoreType.DMA,
            pltpu.SemaphoreType.DMA,
        ],
    )
    return fn(q, k, v, src, dst, zeros_acc)


def kernel(feat, lap_pos, edge_index, We, be, Wl, bl, Wq, Wk, Wv, Wo, bo,
           g1, b1, Wf1, bf1, Wf2, bf2, g2, b2):
    src = edge_index[0]
    dst = edge_index[1]
    sf = jax.random.uniform(jax.random.key(123), (D_LAP,), dtype=jnp.float32)
    sign = jnp.where(sf >= 0.5, 1.0, -1.0)
    zeros_acc = jnp.zeros((N_TOT, D_MODEL), jnp.float32)

    h, q, k, v = _pre(feat, lap_pos, We, be, Wl, bl, sign, Wq[0], Wk[0], Wv[0])
    for l in range(L):
        part = _edge_sc(q, k, v, src, dst, zeros_acc)
        last = l == L - 1
        nq = Wq[l + 1] if not last else Wq[l]
        nk = Wk[l + 1] if not last else Wk[l]
        nv = Wv[l + 1] if not last else Wv[l]
        h, q, k, v = _tail(part, h, Wo[l], bo[l], g1[l], b1[l], Wf1[l], bf1[l],
                           Wf2[l], bf2[l], g2[l], b2[l], nq, nk, nv, last)
    return jnp.mean(h, axis=0, keepdims=True)
